# Initial kernel scaffold; baseline (speedup 1.0000x reference)
import jax
import jax.numpy as jnp
from jax.experimental import pallas as pl


def kernel(x, edge_index, W1, W2, W3):
    raise NotImplementedError("write your pallas kernel here")


# speedup vs baseline: 4.0462x; 4.0462x over previous
import functools

import jax
import jax.numpy as jnp
from jax import lax
from jax.experimental import pallas as pl
from jax.experimental.pallas import tpu as pltpu
from jax.experimental.pallas import tpu_sc as plsc

N = 10000
E = 320000
D = 128
H = 128
C = 6

FT = 4
CE = 6400
NCH = E // CE
GP = CE // 16
ES = E // 16

_mesh = plsc.VectorSubcoreMesh(core_axis_name="c", subcore_axis_name="s")


@functools.partial(
    pl.kernel,
    mesh=_mesh,
    compiler_params=pltpu.CompilerParams(needs_layout_passes=False, use_tc_tiling_on_sc=False),
    out_type=jax.ShapeDtypeStruct((H, N), jnp.float32),
    scratch_types=[
        pltpu.VMEM((FT * N,), jnp.float32),
        pltpu.VMEM((FT * N,), jnp.float32),
        pltpu.VMEM((CE,), jnp.int32),
        pltpu.VMEM((CE,), jnp.int32),
        pltpu.SemaphoreType.DMA,
        pltpu.SemaphoreType.DMA,
    ],
)
def _agg_h(ht, pe, out, hbuf, acc, eb0, eb1, s0, s1):
    wid = lax.axis_index("c") * 16 + lax.axis_index("s")
    base = wid * FT

    pltpu.async_copy(pe.at[pl.ds(0, CE)], eb0, s0)
    pltpu.async_copy(pe.at[pl.ds(CE, CE)], eb1, s1)
    for f in range(FT):
        pltpu.sync_copy(ht.at[base + f], hbuf.at[pl.ds(f * N, N)])

    def zbody(j, _):
        acc[pl.ds(j * 16, 16)] = jnp.zeros((16,), jnp.float32)
        return 0

    lax.fori_loop(0, FT * N // 16, zbody, 0)

    def process(eb):
        def gbody(g, _):
            pe16 = eb[pl.ds(g * 16, 16)]
            src = lax.shift_right_logical(pe16, 16)
            dst = jnp.bitwise_and(pe16, jnp.int32(0xFFFF))
            for f in range(FT):
                vals = plsc.load_gather(hbuf, [src + jnp.int32(f * N)])
                plsc.addupdate_scatter(acc, [dst + jnp.int32(f * N)], vals)
            return 0

        lax.fori_loop(0, GP, gbody, 0)

    def outer(i, _):
        c = 2 * i
        pltpu.make_async_copy(pe.at[pl.ds(0, CE)], eb0, s0).wait()
        process(eb0)

        @pl.when(c + 2 < NCH)
        def _():
            pltpu.async_copy(pe.at[pl.ds((c + 2) * CE, CE)], eb0, s0)

        pltpu.make_async_copy(pe.at[pl.ds(0, CE)], eb1, s1).wait()
        process(eb1)

        @pl.when(c + 3 < NCH)
        def _():
            pltpu.async_copy(pe.at[pl.ds((c + 3) * CE, CE)], eb1, s1)

        return 0

    lax.fori_loop(0, NCH // 2, outer, 0)
    for f in range(FT):
        pltpu.sync_copy(acc.at[pl.ds(f * N, N)], out.at[base + f])


@functools.partial(
    pl.kernel,
    mesh=_mesh,
    compiler_params=pltpu.CompilerParams(needs_layout_passes=False, use_tc_tiling_on_sc=False),
    out_type=jax.ShapeDtypeStruct((32, 3, N), jnp.float32),
    scratch_types=[
        pltpu.VMEM((3 * N,), jnp.float32),
        pltpu.VMEM((3 * N,), jnp.float32),
        pltpu.VMEM((ES,), jnp.int32),
    ],
)
def _agg_out(h3t, pe, out, hbuf, acc, eb):
    wid = lax.axis_index("c") * 16 + lax.axis_index("s")
    grp = wid // 16
    slot = wid % 16

    pltpu.sync_copy(pe.at[pl.ds(slot * ES, ES)], eb)
    for f in range(3):
        pltpu.sync_copy(h3t.at[grp * 3 + f], hbuf.at[pl.ds(f * N, N)])

    def zbody(j, _):
        acc[pl.ds(j * 16, 16)] = jnp.zeros((16,), jnp.float32)
        return 0

    lax.fori_loop(0, 3 * N // 16, zbody, 0)

    def gbody(g, _):
        pe16 = eb[pl.ds(g * 16, 16)]
        src = lax.shift_right_logical(pe16, 16)
        dst = jnp.bitwise_and(pe16, jnp.int32(0xFFFF))
        for f in range(3):
            vals = plsc.load_gather(hbuf, [src + jnp.int32(f * N)])
            plsc.addupdate_scatter(acc, [dst + jnp.int32(f * N)], vals)
        return 0

    lax.fori_loop(0, ES // 16, gbody, 0)
    for f in range(3):
        pltpu.sync_copy(acc.at[pl.ds(f * N, N)], out.at[wid, f])


def _pack_body(e_ref, o_ref):
    s = e_ref[0, :]
    d = e_ref[1, :]
    o_ref[...] = jnp.bitwise_or(lax.shift_left(s, 16), d)


def _pack_edges(edge_index):
    return pl.pallas_call(
        _pack_body,
        out_shape=jax.ShapeDtypeStruct((E,), jnp.int32),
    )(edge_index)


def _mm_rows_body(x_ref, w_ref, o_ref):
    o_ref[...] = lax.dot_general(
        w_ref[...], x_ref[...], (((0,), (1,)), ((), ())),
        preferred_element_type=jnp.float32)


def _mm_rows(x, w):
    do = w.shape[1]
    return pl.pallas_call(
        _mm_rows_body,
        out_shape=jax.ShapeDtypeStruct((do, N), jnp.float32),
    )(x, w)


def _mm_t_body(h_ref, w_ref, o_ref):
    o_ref[...] = lax.dot_general(
        w_ref[...], h_ref[...], (((0,), (0,)), ((), ())),
        preferred_element_type=jnp.float32)


def _mm_t(ht, w):
    do = w.shape[1]
    return pl.pallas_call(
        _mm_t_body,
        out_shape=jax.ShapeDtypeStruct((do, N), jnp.float32),
    )(ht, w)


def _comb_body(p_ref, o_ref):
    p = p_ref[...]
    s0 = jnp.sum(p[0:16], axis=0)
    s1 = jnp.sum(p[16:32], axis=0)
    s6 = jnp.concatenate([s0, s1], axis=0)
    i6 = (lax.broadcasted_iota(jnp.int32, (C, C), 0)
          == lax.broadcasted_iota(jnp.int32, (C, C), 1)).astype(jnp.float32)
    o_ref[...] = lax.dot_general(
        s6, i6, (((0,), (0,)), ((), ())), preferred_element_type=jnp.float32)


def _combine(parts):
    return pl.pallas_call(
        _comb_body,
        out_shape=jax.ShapeDtypeStruct((N, C), jnp.float32),
    )(parts)


def kernel(x, edge_index, W1, W2, W3):
    pe = _pack_edges(edge_index)
    h1t = _mm_rows(x, W1)
    a1t = _agg_h(h1t, pe)
    h2t = _mm_t(a1t, W2)
    a2t = _agg_h(h2t, pe)
    h3t = _mm_t(a2t, W3)
    parts = _agg_out(h3t, pe)
    return _combine(parts)


# speedup vs baseline: 10.7007x; 2.6446x over previous
import functools

import jax
import jax.numpy as jnp
from jax import lax
from jax.experimental import pallas as pl
from jax.experimental.pallas import tpu as pltpu
from jax.experimental.pallas import tpu_sc as plsc

N = 10000
E = 320000
D = 128
H = 128
C = 6

FT = 4
CE = 6400
NCH = E // CE
GP = CE // 16
ES = E // 16

_mesh = plsc.VectorSubcoreMesh(core_axis_name="c", subcore_axis_name="s")


@functools.partial(
    pl.kernel,
    mesh=_mesh,
    compiler_params=pltpu.CompilerParams(needs_layout_passes=False, use_tc_tiling_on_sc=False, disable_bounds_checks=True),
    out_type=jax.ShapeDtypeStruct((H, N), jnp.float32),
    scratch_types=[
        pltpu.VMEM((FT * N,), jnp.float32),
        pltpu.VMEM((FT * N,), jnp.float32),
        pltpu.VMEM((CE,), jnp.int32),
        pltpu.VMEM((CE,), jnp.int32),
        pltpu.SemaphoreType.DMA,
        pltpu.SemaphoreType.DMA,
    ],
)
def _agg_h(ht, pe, out, hbuf, acc, eb0, eb1, s0, s1):
    wid = lax.axis_index("c") * 16 + lax.axis_index("s")
    base = wid * FT

    pltpu.async_copy(pe.at[pl.ds(0, CE)], eb0, s0)
    pltpu.async_copy(pe.at[pl.ds(CE, CE)], eb1, s1)
    for f in range(FT):
        pltpu.sync_copy(ht.at[base + f], hbuf.at[pl.ds(f * N, N)])

    @plsc.parallel_loop(0, FT * N // 16, unroll=8)
    def _zero(j):
        acc[pl.ds(j * 16, 16)] = jnp.zeros((16,), jnp.float32)

    def process(eb):
        @plsc.parallel_loop(0, GP, unroll=8)
        def _gather_scatter(g):
            pe16 = eb[pl.ds(g * 16, 16)]
            src = lax.shift_right_logical(pe16, 16)
            dst = jnp.bitwise_and(pe16, jnp.int32(0xFFFF))
            for f in range(FT):
                vals = plsc.load_gather(hbuf, [src + jnp.int32(f * N)])
                plsc.addupdate_scatter(acc, [dst + jnp.int32(f * N)], vals)

    def outer(i, _):
        c = 2 * i
        pltpu.make_async_copy(pe.at[pl.ds(0, CE)], eb0, s0).wait()
        process(eb0)

        @pl.when(c + 2 < NCH)
        def _():
            pltpu.async_copy(pe.at[pl.ds((c + 2) * CE, CE)], eb0, s0)

        pltpu.make_async_copy(pe.at[pl.ds(0, CE)], eb1, s1).wait()
        process(eb1)

        @pl.when(c + 3 < NCH)
        def _():
            pltpu.async_copy(pe.at[pl.ds((c + 3) * CE, CE)], eb1, s1)

        return 0

    lax.fori_loop(0, NCH // 2, outer, 0)
    for f in range(FT):
        pltpu.sync_copy(acc.at[pl.ds(f * N, N)], out.at[base + f])


@functools.partial(
    pl.kernel,
    mesh=_mesh,
    compiler_params=pltpu.CompilerParams(needs_layout_passes=False, use_tc_tiling_on_sc=False, disable_bounds_checks=True),
    out_type=jax.ShapeDtypeStruct((32, 3, N), jnp.float32),
    scratch_types=[
        pltpu.VMEM((3 * N,), jnp.float32),
        pltpu.VMEM((3 * N,), jnp.float32),
        pltpu.VMEM((ES,), jnp.int32),
    ],
)
def _agg_out(h3t, pe, out, hbuf, acc, eb):
    wid = lax.axis_index("c") * 16 + lax.axis_index("s")
    grp = wid // 16
    slot = wid % 16

    pltpu.sync_copy(pe.at[pl.ds(slot * ES, ES)], eb)
    for f in range(3):
        pltpu.sync_copy(h3t.at[grp * 3 + f], hbuf.at[pl.ds(f * N, N)])

    @plsc.parallel_loop(0, 3 * N // 16, unroll=8)
    def _zero(j):
        acc[pl.ds(j * 16, 16)] = jnp.zeros((16,), jnp.float32)

    @plsc.parallel_loop(0, ES // 16, unroll=8)
    def _gather_scatter(g):
        pe16 = eb[pl.ds(g * 16, 16)]
        src = lax.shift_right_logical(pe16, 16)
        dst = jnp.bitwise_and(pe16, jnp.int32(0xFFFF))
        for f in range(3):
            vals = plsc.load_gather(hbuf, [src + jnp.int32(f * N)])
            plsc.addupdate_scatter(acc, [dst + jnp.int32(f * N)], vals)
    for f in range(3):
        pltpu.sync_copy(acc.at[pl.ds(f * N, N)], out.at[wid, f])


def _pack_body(e_ref, o_ref):
    s = e_ref[0, :]
    d = e_ref[1, :]
    o_ref[...] = jnp.bitwise_or(lax.shift_left(s, 16), d)


def _pack_edges(edge_index):
    return pl.pallas_call(
        _pack_body,
        out_shape=jax.ShapeDtypeStruct((E,), jnp.int32),
    )(edge_index)


def _mm_rows_body(x_ref, w_ref, o_ref):
    o_ref[...] = lax.dot_general(
        w_ref[...], x_ref[...], (((0,), (1,)), ((), ())),
        preferred_element_type=jnp.float32)


def _mm_rows(x, w):
    do = w.shape[1]
    return pl.pallas_call(
        _mm_rows_body,
        out_shape=jax.ShapeDtypeStruct((do, N), jnp.float32),
    )(x, w)


def _mm_t_body(h_ref, w_ref, o_ref):
    o_ref[...] = lax.dot_general(
        w_ref[...], h_ref[...], (((0,), (0,)), ((), ())),
        preferred_element_type=jnp.float32)


def _mm_t(ht, w):
    do = w.shape[1]
    return pl.pallas_call(
        _mm_t_body,
        out_shape=jax.ShapeDtypeStruct((do, N), jnp.float32),
    )(ht, w)


def _comb_body(p_ref, o_ref):
    p = p_ref[...]
    s0 = jnp.sum(p[0:16], axis=0)
    s1 = jnp.sum(p[16:32], axis=0)
    s6 = jnp.concatenate([s0, s1], axis=0)
    i6 = (lax.broadcasted_iota(jnp.int32, (C, C), 0)
          == lax.broadcasted_iota(jnp.int32, (C, C), 1)).astype(jnp.float32)
    o_ref[...] = lax.dot_general(
        s6, i6, (((0,), (0,)), ((), ())), preferred_element_type=jnp.float32)


def _combine(parts):
    return pl.pallas_call(
        _comb_body,
        out_shape=jax.ShapeDtypeStruct((N, C), jnp.float32),
    )(parts)


def kernel(x, edge_index, W1, W2, W3):
    pe = _pack_edges(edge_index)
    h1t = _mm_rows(x, W1)
    a1t = _agg_h(h1t, pe)
    h2t = _mm_t(a1t, W2)
    a2t = _agg_h(h2t, pe)
    h3t = _mm_t(a2t, W3)
    parts = _agg_out(h3t, pe)
    return _combine(parts)
